# Initial kernel scaffold; baseline (speedup 1.0000x reference)
import jax
import jax.numpy as jnp
from jax.experimental import pallas as pl


def kernel(clauses_batch, var_embed, sign_embed, lin1_W, lin1_b, lin2_W, lin2_b, cn_g, cn_b, prefix_queries, in_proj_w, in_proj_b, out_proj_w, out_proj_b, pn_g, pn_b):
    raise NotImplementedError("write your pallas kernel here")


# speedup vs baseline: 8.2311x; 8.2311x over previous
import functools
import math

import jax
import jax.numpy as jnp
from jax import lax
from jax.experimental import pallas as pl
from jax.experimental.pallas import tpu as pltpu
from jax.experimental.pallas import tpu_sc as plsc

D = 128
HEADS = 8
P = 32
B, C, L = 8, 2048, 8
MAX_LIT = 256
EPS = 1e-5
VPAD = 264
T = 2 * VPAD


def _gelu(x):
    return 0.5 * x * (1.0 + lax.erf(x * (1.0 / math.sqrt(2.0))))


def _table_body(vp_ref, se_ref, w1v_ref, w1s_ref, b1_ref, w2_ref, b2_ref, out_ref):
    pv = jnp.dot(vp_ref[...], w1v_ref[...], preferred_element_type=jnp.float32)
    ps = jnp.dot(se_ref[...], w1s_ref[...], preferred_element_type=jnp.float32)
    for s in range(2):
        pre = pv + ps[s:s + 1, :] + b1_ref[...]
        h = _gelu(pre)
        out_ref[s * VPAD:(s + 1) * VPAD, :] = (
            jnp.dot(h, w2_ref[...], preferred_element_type=jnp.float32) + b2_ref[...]
        )


def _build_table(var_pad, sign_embed, w1v_t, w1s_t, b1, w2_t, b2, interpret=False):
    return pl.pallas_call(
        _table_body,
        out_shape=jax.ShapeDtypeStruct((T, D), jnp.float32),
        interpret=interpret,
    )(var_pad, sign_embed, w1v_t, w1s_t, b1, w2_t, b2)


def _attn_body(cs_ref, pq_ref, wq_ref, wk_ref, wv_ref, bq_ref, bk_ref, bv_ref,
               cng_ref, cnb_ref, wo_ref, bo_ref, png_ref, pnb_ref, out_ref):
    dh = D // HEADS
    cs = cs_ref[...] * (1.0 / L)
    mu = jnp.mean(cs, axis=-1, keepdims=True)
    var = jnp.mean((cs - mu) ** 2, axis=-1, keepdims=True)
    ce = (cs - mu) * lax.rsqrt(var + EPS) * cng_ref[...] + cnb_ref[...]

    k = jnp.dot(ce, wk_ref[...], preferred_element_type=jnp.float32) + bk_ref[...]
    v = jnp.dot(ce, wv_ref[...], preferred_element_type=jnp.float32) + bv_ref[...]
    pq = pq_ref[...]
    q = jnp.dot(pq, wq_ref[...], preferred_element_type=jnp.float32) + bq_ref[...]

    qe = jnp.broadcast_to(q[None], (HEADS, P, D)).reshape(HEADS * P, D)
    row = lax.broadcasted_iota(jnp.int32, (HEADS * P, D), 0)
    col = lax.broadcasted_iota(jnp.int32, (HEADS * P, D), 1)
    hm = ((col // dh) == (row // P)).astype(jnp.float32)
    qm = qe * hm

    scores = lax.dot_general(qm, k, dimension_numbers=(((1,), (1,)), ((), ())),
                             preferred_element_type=jnp.float32)
    scores = scores * (1.0 / math.sqrt(dh))
    mx = jnp.max(scores, axis=-1, keepdims=True)
    e = jnp.exp(scores - mx)
    attn = e / jnp.sum(e, axis=-1, keepdims=True)

    ctxh = jnp.dot(attn, v, preferred_element_type=jnp.float32)
    ctx = jnp.sum((ctxh * hm).reshape(HEADS, P, D), axis=0)

    refined = jnp.dot(ctx, wo_ref[...], preferred_element_type=jnp.float32) + bo_ref[...]
    x = pq + refined
    mu2 = jnp.mean(x, axis=-1, keepdims=True)
    var2 = jnp.mean((x - mu2) ** 2, axis=-1, keepdims=True)
    out_ref[0] = (x - mu2) * lax.rsqrt(var2 + EPS) * png_ref[...] + pnb_ref[...]


def _attention(clause_sum, pq, wq_t, wk_t, wv_t, bq, bk, bv, cn_g, cn_b,
               wo_t, bo, pn_g, pn_b, interpret=False):
    rep = pl.BlockSpec(None, lambda b: (0,) * 2)
    return pl.pallas_call(
        _attn_body,
        grid=(B,),
        in_specs=[
            pl.BlockSpec((C, D), lambda b: (b, 0)),
            rep, rep, rep, rep, rep, rep, rep, rep, rep, rep, rep, rep, rep,
        ],
        out_specs=pl.BlockSpec((1, P, D), lambda b: (b, 0, 0)),
        out_shape=jax.ShapeDtypeStruct((B, P, D), jnp.float32),
        interpret=interpret,
    )(clause_sum, pq, wq_t, wk_t, wv_t, bq, bk, bv, cn_g, cn_b, wo_t, bo, pn_g, pn_b)


NW = 32
NCL_W = B * C // NW
NLIT_W = NCL_W * L
CHW = 128
NCH = NLIT_W // CHW
ROWS_CH = CHW


def _sc_body(cl_hbm, table_hbm, zeros_hbm, out_hbm, cl_v, dst_v, rows_v, acc_sh, sem):
    cid = lax.axis_index("c")
    sid = lax.axis_index("s")
    wid = sid * 2 + cid
    pltpu.sync_copy(cl_hbm.at[wid], cl_v)
    pltpu.sync_copy(zeros_hbm, acc_sh.at[pl.ds(sid * NCL_W, NCL_W)])

    def prep(r, carry):
        io = lax.iota(jnp.int32, 16)
        for c8 in range(8):
            x = cl_v[r, pl.ds(c8 * 16, 16)]
            lit = jnp.minimum(jnp.abs(x), MAX_LIT)
            comb = jnp.where(x > 0, lit + VPAD, lit)
            cl_v[r, pl.ds(c8 * 16, 16)] = comb
            dst_v[r, pl.ds(c8 * 16, 16)] = (sid * NCL_W + r * 16 + 2 * c8
                                            + lax.shift_right_arithmetic(io, 3))
        return carry

    lax.fori_loop(0, NCH, prep, 0)

    def chunk(j, carry):
        pltpu.async_copy(table_hbm.at[cl_v.at[j]], rows_v, sem).wait()
        pltpu.sync_copy(rows_v, acc_sh.at[dst_v.at[j]], add=True)
        return carry

    lax.fori_loop(0, NCH, chunk, 0)
    pltpu.sync_copy(acc_sh.at[pl.ds(sid * NCL_W, NCL_W)],
                    out_hbm.at[pl.ds(wid * NCL_W, NCL_W)])


def _gather_mean(clauses_flat, table):
    cl3 = clauses_flat.reshape(NW, NCH, CHW)
    zeros = jnp.zeros((NCL_W, D), jnp.float32)
    mesh = plsc.VectorSubcoreMesh(core_axis_name="c", subcore_axis_name="s")
    f = pl.kernel(
        _sc_body,
        out_type=jax.ShapeDtypeStruct((B * C, D), jnp.float32),
        mesh=mesh,
        scratch_types=[
            pltpu.VMEM((NCH, CHW), jnp.int32),
            pltpu.VMEM((NCH, CHW), jnp.int32),
            pltpu.VMEM((ROWS_CH, D), jnp.float32),
            pltpu.VMEM_SHARED((16 * NCL_W, D), jnp.float32),
            pltpu.SemaphoreType.DMA,
        ],
    )
    return f(cl3, table, zeros)


def kernel(clauses_batch, var_embed, sign_embed, lin1_W, lin1_b, lin2_W, lin2_b,
           cn_g, cn_b, prefix_queries, in_proj_w, in_proj_b, out_proj_w,
           out_proj_b, pn_g, pn_b, _interpret=False):
    f32 = jnp.float32
    var_pad = jnp.zeros((VPAD, D), f32).at[:MAX_LIT + 1].set(var_embed)
    w1v_t = lin1_W[:, :D].T
    w1s_t = lin1_W[:, D:].T
    table = _build_table(var_pad, sign_embed, w1v_t, w1s_t,
                         lin1_b.reshape(1, D), lin2_W.T, lin2_b.reshape(1, D),
                         interpret=_interpret)

    clauses_flat = clauses_batch.reshape(B * C * L)
    clause_sum = _gather_mean(clauses_flat, table)

    wq_t = in_proj_w[:D].T
    wk_t = in_proj_w[D:2 * D].T
    wv_t = in_proj_w[2 * D:].T
    bq = in_proj_b[:D].reshape(1, D)
    bk = in_proj_b[D:2 * D].reshape(1, D)
    bv = in_proj_b[2 * D:].reshape(1, D)
    return _attention(clause_sum, prefix_queries, wq_t, wk_t, wv_t, bq, bk, bv,
                      cn_g.reshape(1, D), cn_b.reshape(1, D),
                      out_proj_w.T, out_proj_b.reshape(1, D),
                      pn_g.reshape(1, D), pn_b.reshape(1, D),
                      interpret=_interpret)


# speedup vs baseline: 8.3843x; 1.0186x over previous
import functools
import math

import jax
import jax.numpy as jnp
from jax import lax
from jax.experimental import pallas as pl
from jax.experimental.pallas import tpu as pltpu
from jax.experimental.pallas import tpu_sc as plsc

D = 128
HEADS = 8
P = 32
B, C, L = 8, 2048, 8
MAX_LIT = 256
EPS = 1e-5
VPAD = 264
T = 2 * VPAD


def _gelu(x):
    return 0.5 * x * (1.0 + lax.erf(x * (1.0 / math.sqrt(2.0))))


def _table_body(vp_ref, se_ref, w1v_ref, w1s_ref, b1_ref, w2_ref, b2_ref, out_ref):
    pv = jnp.dot(vp_ref[...], w1v_ref[...], preferred_element_type=jnp.float32)
    ps = jnp.dot(se_ref[...], w1s_ref[...], preferred_element_type=jnp.float32)
    for s in range(2):
        pre = pv + ps[s:s + 1, :] + b1_ref[...]
        h = _gelu(pre)
        out_ref[s * VPAD:(s + 1) * VPAD, :] = (
            jnp.dot(h, w2_ref[...], preferred_element_type=jnp.float32) + b2_ref[...]
        )


def _build_table(var_pad, sign_embed, w1v_t, w1s_t, b1, w2_t, b2, interpret=False):
    return pl.pallas_call(
        _table_body,
        out_shape=jax.ShapeDtypeStruct((T, D), jnp.float32),
        interpret=interpret,
    )(var_pad, sign_embed, w1v_t, w1s_t, b1, w2_t, b2)


def _attn_body(cs_ref, pq_ref, wq_ref, wk_ref, wv_ref, bq_ref, bk_ref, bv_ref,
               cng_ref, cnb_ref, wo_ref, bo_ref, png_ref, pnb_ref, out_ref):
    dh = D // HEADS
    cs = cs_ref[...] * (1.0 / L)
    mu = jnp.mean(cs, axis=-1, keepdims=True)
    var = jnp.mean((cs - mu) ** 2, axis=-1, keepdims=True)
    ce = (cs - mu) * lax.rsqrt(var + EPS) * cng_ref[...] + cnb_ref[...]

    k = jnp.dot(ce, wk_ref[...], preferred_element_type=jnp.float32) + bk_ref[...]
    v = jnp.dot(ce, wv_ref[...], preferred_element_type=jnp.float32) + bv_ref[...]
    pq = pq_ref[...]
    q = jnp.dot(pq, wq_ref[...], preferred_element_type=jnp.float32) + bq_ref[...]

    qe = jnp.broadcast_to(q[None], (HEADS, P, D)).reshape(HEADS * P, D)
    row = lax.broadcasted_iota(jnp.int32, (HEADS * P, D), 0)
    col = lax.broadcasted_iota(jnp.int32, (HEADS * P, D), 1)
    hm = ((col // dh) == (row // P)).astype(jnp.float32)
    qm = qe * hm

    scores = lax.dot_general(qm, k, dimension_numbers=(((1,), (1,)), ((), ())),
                             preferred_element_type=jnp.float32)
    scores = scores * (1.0 / math.sqrt(dh))
    mx = jnp.max(scores, axis=-1, keepdims=True)
    e = jnp.exp(scores - mx)
    attn = e / jnp.sum(e, axis=-1, keepdims=True)

    ctxh = jnp.dot(attn, v, preferred_element_type=jnp.float32)
    ctx = jnp.sum((ctxh * hm).reshape(HEADS, P, D), axis=0)

    refined = jnp.dot(ctx, wo_ref[...], preferred_element_type=jnp.float32) + bo_ref[...]
    x = pq + refined
    mu2 = jnp.mean(x, axis=-1, keepdims=True)
    var2 = jnp.mean((x - mu2) ** 2, axis=-1, keepdims=True)
    out_ref[0] = (x - mu2) * lax.rsqrt(var2 + EPS) * png_ref[...] + pnb_ref[...]


def _attention(clause_sum, pq, wq_t, wk_t, wv_t, bq, bk, bv, cn_g, cn_b,
               wo_t, bo, pn_g, pn_b, interpret=False):
    rep = pl.BlockSpec(None, lambda b: (0,) * 2)
    return pl.pallas_call(
        _attn_body,
        grid=(B,),
        in_specs=[
            pl.BlockSpec((C, D), lambda b: (b, 0)),
            rep, rep, rep, rep, rep, rep, rep, rep, rep, rep, rep, rep, rep,
        ],
        out_specs=pl.BlockSpec((1, P, D), lambda b: (b, 0, 0)),
        out_shape=jax.ShapeDtypeStruct((B, P, D), jnp.float32),
        interpret=interpret,
    )(clause_sum, pq, wq_t, wk_t, wv_t, bq, bk, bv, cn_g, cn_b, wo_t, bo, pn_g, pn_b)


NW = 32
NCL_W = B * C // NW
NLIT_W = NCL_W * L
CHW = 128
NCH = NLIT_W // CHW
ROWS_CH = CHW


def _sc_body(cl_hbm, table_hbm, zeros_hbm, out_hbm, cl_v, dst_v, rows_v, acc_sh, sem, sem2):
    cid = lax.axis_index("c")
    sid = lax.axis_index("s")
    wid = sid * 2 + cid
    pltpu.sync_copy(cl_hbm.at[wid], cl_v)
    pltpu.sync_copy(zeros_hbm, acc_sh.at[pl.ds(sid * NCL_W, NCL_W)])

    def prep(r, carry):
        io = lax.iota(jnp.int32, 16)
        for c8 in range(8):
            x = cl_v[r, pl.ds(c8 * 16, 16)]
            lit = jnp.minimum(jnp.abs(x), MAX_LIT)
            comb = jnp.where(x > 0, lit + VPAD, lit)
            cl_v[r, pl.ds(c8 * 16, 16)] = comb
            dst_v[r, pl.ds(c8 * 16, 16)] = (sid * NCL_W + r * 16 + 2 * c8
                                            + lax.shift_right_arithmetic(io, 3))
        return carry

    lax.fori_loop(0, NCH, prep, 0)

    sems = (sem, sem2)
    desc = pltpu.async_copy(table_hbm.at[cl_v.at[0]], rows_v.at[0], sems[0])
    for j in range(NCH):
        b = j & 1
        desc.wait()
        if j + 1 < NCH:
            desc = pltpu.async_copy(table_hbm.at[cl_v.at[j + 1]],
                                    rows_v.at[1 - b], sems[1 - b])
        pltpu.sync_copy(rows_v.at[b], acc_sh.at[dst_v.at[j]], add=True)
    pltpu.sync_copy(acc_sh.at[pl.ds(sid * NCL_W, NCL_W)],
                    out_hbm.at[pl.ds(wid * NCL_W, NCL_W)])


def _gather_mean(clauses_flat, table):
    cl3 = clauses_flat.reshape(NW, NCH, CHW)
    zeros = jnp.zeros((NCL_W, D), jnp.float32)
    mesh = plsc.VectorSubcoreMesh(core_axis_name="c", subcore_axis_name="s")
    f = pl.kernel(
        _sc_body,
        out_type=jax.ShapeDtypeStruct((B * C, D), jnp.float32),
        mesh=mesh,
        scratch_types=[
            pltpu.VMEM((NCH, CHW), jnp.int32),
            pltpu.VMEM((NCH, CHW), jnp.int32),
            pltpu.VMEM((2, ROWS_CH, D), jnp.float32),
            pltpu.VMEM_SHARED((16 * NCL_W, D), jnp.float32),
            pltpu.SemaphoreType.DMA,
            pltpu.SemaphoreType.DMA,
        ],
    )
    return f(cl3, table, zeros)


def kernel(clauses_batch, var_embed, sign_embed, lin1_W, lin1_b, lin2_W, lin2_b,
           cn_g, cn_b, prefix_queries, in_proj_w, in_proj_b, out_proj_w,
           out_proj_b, pn_g, pn_b, _interpret=False):
    f32 = jnp.float32
    var_pad = jnp.zeros((VPAD, D), f32).at[:MAX_LIT + 1].set(var_embed)
    w1v_t = lin1_W[:, :D].T
    w1s_t = lin1_W[:, D:].T
    table = _build_table(var_pad, sign_embed, w1v_t, w1s_t,
                         lin1_b.reshape(1, D), lin2_W.T, lin2_b.reshape(1, D),
                         interpret=_interpret)

    clauses_flat = clauses_batch.reshape(B * C * L)
    clause_sum = _gather_mean(clauses_flat, table)

    wq_t = in_proj_w[:D].T
    wk_t = in_proj_w[D:2 * D].T
    wv_t = in_proj_w[2 * D:].T
    bq = in_proj_b[:D].reshape(1, D)
    bk = in_proj_b[D:2 * D].reshape(1, D)
    bv = in_proj_b[2 * D:].reshape(1, D)
    return _attention(clause_sum, prefix_queries, wq_t, wk_t, wv_t, bq, bk, bv,
                      cn_g.reshape(1, D), cn_b.reshape(1, D),
                      out_proj_w.T, out_proj_b.reshape(1, D),
                      pn_g.reshape(1, D), pn_b.reshape(1, D),
                      interpret=_interpret)
